# Initial kernel scaffold; baseline (speedup 1.0000x reference)
import jax
import jax.numpy as jnp
from jax.experimental import pallas as pl


def kernel(x, edgeA, edgeB, edge_feat, W_in, b_in, Wu, au, Wv, av, Wou, aou, Wov, aov, W1, b1, W2, b2, W3):
    raise NotImplementedError("write your pallas kernel here")


# speedup vs baseline: 8.4536x; 8.4536x over previous
import functools

import jax
import jax.numpy as jnp
from jax import lax
from jax.experimental import pallas as pl
from jax.experimental.pallas import tpu as pltpu
from jax.experimental.pallas import tpu_sc as plsc

N = 10000
NPAD = 10240
E = 320000
EPAD = 327680
NC = 2
NS = 16
RPT = NPAD // NS
ALPHA = 0.2


def _make_edge_pass(K):
    ECHUNK = EPAD // NS if K == 2 else EPAD // (NC * NS)
    NWIN = ECHUNK // 128
    RSN = NPAD * K

    mesh = plsc.VectorSubcoreMesh(core_axis_name="c", subcore_axis_name="s")

    out_type = [jax.ShapeDtypeStruct((NC, NPAD, 128), jnp.float32),
                jax.ShapeDtypeStruct((NC, RSN), jnp.float32)]
    if K == 2:
        out_type.append(jax.ShapeDtypeStruct((NC, EPAD), jnp.float32))

    scratch = [
        pltpu.VMEM_SHARED((NPAD, 128), jnp.float32),
        pltpu.VMEM_SHARED((RSN,), jnp.float32),
        [pltpu.VMEM_SHARED((NPAD,), jnp.float32) for _ in range(2 * K)],
        pltpu.VMEM((128,), jnp.int32),
        pltpu.VMEM((128,), jnp.int32),
        pltpu.VMEM((128,), jnp.float32),
        pltpu.VMEM((128, 128), jnp.float32),
        [pltpu.VMEM((128,), jnp.float32) for _ in range(2 * K)],
        pltpu.VMEM((128, 128), jnp.float32),
        [pltpu.VMEM((128,), jnp.float32) for _ in range(K)],
        [pltpu.VMEM((128,), jnp.int32) for _ in range(K)],
        pltpu.VMEM((128,), jnp.float32),
        pltpu.VMEM((2, 16), jnp.float32),
        pltpu.SemaphoreType.DMA,
    ]

    @functools.partial(pl.kernel, out_type=out_type, scratch_types=scratch,
                       mesh=mesh)
    def edge_pass(src_hbm, dst_hbm, ef_hbm, g_hbm, ss_hbm, cb_hbm,
                  z2_hbm, z1_hbm, m_out, rs_out, *rest):
        if K == 2:
            ep_out = rest[0]
            rest = rest[1:]
        (m_sp, rs_sp, ss_sps, src_v, dst_v, ef_v, g_v, ss_vs, pay_v,
         e_vs, ix_vs, ep_v, cb_v, sem) = rest

        c = lax.axis_index("c")
        s = lax.axis_index("s")
        r0 = s * RPT

        for t in range(2 * K):
            pltpu.sync_copy(ss_hbm.at[c, t, pl.ds(r0, RPT)],
                            ss_sps[t].at[pl.ds(r0, RPT)])
        for j in range(5):
            pltpu.sync_copy(z2_hbm.at[pl.ds(0, 128)],
                            m_sp.at[pl.ds(r0 + j * 128, 128)])
        pltpu.sync_copy(z1_hbm.at[pl.ds(0, RSN // NS)],
                        rs_sp.at[pl.ds(s * (RSN // NS), RSN // NS)])
        pltpu.sync_copy(z2_hbm.at[pl.ds(0, 128)], pay_v)
        pltpu.sync_copy(cb_hbm.at[c], cb_v)
        plsc.subcore_barrier()

        cvals = [cb_v[h] for h in range(K)]

        if K == 2:
            wbase = s * ECHUNK
        else:
            wbase = (c * NS + s) * ECHUNK

        iota16 = lax.iota(jnp.int32, 16)

        def window(w, carry):
            eb = wbase + w * 128
            pltpu.sync_copy(src_hbm.at[pl.ds(eb, 128)], src_v)
            pltpu.sync_copy(dst_hbm.at[pl.ds(eb, 128)], dst_v)
            pltpu.sync_copy(ef_hbm.at[pl.ds(eb, 128)], ef_v)
            pltpu.async_copy(g_hbm.at[dst_v], g_v, sem).wait()
            for t in range(K):
                pltpu.async_copy(ss_sps[t].at[src_v], ss_vs[t], sem).wait()
                pltpu.async_copy(ss_sps[K + t].at[dst_v], ss_vs[K + t],
                                 sem).wait()

            for g16 in range(8):
                o = g16 * 16
                efv = ef_v[pl.ds(o, 16)]
                gid = (eb + o) + iota16
                valid = gid < E
                esum = jnp.zeros((16,), jnp.float32)
                evals = []
                if K == 2:
                    src16 = src_v[pl.ds(o, 16)]
                for h in range(K):
                    s1 = ss_vs[h][pl.ds(o, 16)]
                    s2 = ss_vs[K + h][pl.ds(o, 16)]
                    lg = s1 + s2 + cvals[h] * efv
                    e = jnp.exp(-jnp.maximum(lg, ALPHA * lg))
                    e = jnp.where(valid, e, jnp.zeros((16,), jnp.float32))
                    evals.append(e)
                    esum = esum + e
                    e_vs[h][pl.ds(o, 16)] = e
                    if K == 2:
                        ix_vs[h][pl.ds(o, 16)] = src16 * 2 + h
                if K == 2:
                    ep_v[pl.ds(o, 16)] = esum
                for j in range(16):
                    w2 = o + j
                    gq = [g_v[w2, pl.ds(q * 16, 16)] for q in range(4)]
                    for h in range(K):
                        ev = jnp.full((16,), evals[h][j])
                        for q in range(4):
                            pay_v[w2, pl.ds(h * 64 + q * 16, 16)] = ev * gq[q]

            pltpu.sync_copy(pay_v, m_sp.at[src_v], add=True)
            if K == 2:
                for h in range(K):
                    pltpu.sync_copy(e_vs[h], rs_sp.at[ix_vs[h]], add=True)
                pltpu.sync_copy(ep_v, ep_out.at[c, pl.ds(eb, 128)])
            else:
                pltpu.sync_copy(e_vs[0], rs_sp.at[src_v], add=True)
            return carry

        lax.fori_loop(0, NWIN, window, 0)
        plsc.subcore_barrier()
        pltpu.sync_copy(m_sp.at[pl.ds(r0, RPT)], m_out.at[c, pl.ds(r0, RPT)])
        pltpu.sync_copy(rs_sp.at[pl.ds(s * (RSN // NS), RSN // NS)],
                        rs_out.at[c, pl.ds(s * (RSN // NS), RSN // NS)])

    return edge_pass


_edge_multi = _make_edge_pass(2)
_edge_single = _make_edge_pass(1)


RB = 2048


def _node_call(body, nout, out_widths, in_specs, out_dtype=jnp.float32):
    grid = (NPAD // RB,)
    out_specs = [pl.BlockSpec((RB, w), lambda i: (i, 0)) for w in out_widths]
    out_shape = [jax.ShapeDtypeStruct((NPAD, w), out_dtype)
                 for w in out_widths]
    if nout == 1:
        out_specs, out_shape = out_specs[0], out_shape[0]
    return pl.pallas_call(body, grid=grid, in_specs=in_specs,
                          out_specs=out_specs, out_shape=out_shape)


def _full(shape):
    return pl.BlockSpec(shape, lambda i: tuple(0 for _ in shape))


def _tc_stage0(x, W_in, b_in, PQ):
    def body(x_r, w_r, b_r, pq_r, h_r, ss_r):
        h = jnp.dot(x_r[...], w_r[...],
                    preferred_element_type=jnp.float32) + b_r[...]
        h_r[...] = jnp.concatenate([h, jnp.zeros((RB, 64), jnp.float32)], 1)
        ss_r[...] = jnp.dot(h, pq_r[...], preferred_element_type=jnp.float32)
    specs = [pl.BlockSpec((RB, 128), lambda i: (i, 0)), _full((128, 64)),
             _full((1, 64)), _full((64, 8))]
    return _node_call(body, 2, [128, 8], specs)(x, W_in, b_in, PQ)


def _tc_heads(M, RS, Wh, Wo, PQ):
    def body(m_r, rs_r, wh_r, wo_r, pq_r, g_r, ss_r):
        m = m_r[...]
        r = rs_r[...]
        wh = wh_r[...]
        wo = wo_r[...]
        acc = jnp.zeros((RB, 64), jnp.float32)
        for i in range(4):
            ci, li = divmod(i, 2)
            Mi = m[ci, :, li * 64:(li + 1) * 64]
            rs = r[ci, :, li][:, None]
            hp = jnp.dot(Mi / (rs + 1e-16), wh[i],
                         preferred_element_type=jnp.float32)
            el = jnp.where(hp > 0, hp, jnp.exp(hp) - 1.0)
            acc = acc + jnp.dot(el, wo[i * 64:(i + 1) * 64],
                                preferred_element_type=jnp.float32)
        g_r[...] = jnp.concatenate([acc, jnp.zeros((RB, 64), jnp.float32)], 1)
        ss_r[...] = jnp.dot(acc, pq_r[...], preferred_element_type=jnp.float32)
    specs = [pl.BlockSpec((2, RB, 128), lambda i: (0, i, 0)),
             pl.BlockSpec((2, RB, 2), lambda i: (0, i, 0)),
             _full((4, 64, 64)), _full((256, 64)), _full((64, 8))]
    return _node_call(body, 2, [128, 8], specs)(M, RS, Wh, Wo, PQ)


def _tc_combine(MO, RS, PQ):
    def body(m_r, rs_r, pq_r, h_r, ss_r):
        m = m_r[...]
        r = rs_r[...]
        num = m[0, :, :64] + m[1, :, :64]
        rs = (r[0, :, 0] + r[1, :, 0])[:, None]
        h = num / (rs + 1e-16)
        h_r[...] = jnp.concatenate([h, jnp.zeros((RB, 64), jnp.float32)], 1)
        ss_r[...] = jnp.dot(h, pq_r[...], preferred_element_type=jnp.float32)
    specs = [pl.BlockSpec((2, RB, 128), lambda i: (0, i, 0)),
             pl.BlockSpec((2, RB, 1), lambda i: (0, i, 0)), _full((64, 8))]
    return _node_call(body, 2, [128, 8], specs)(MO, RS, PQ)


def _tc_final(MO, RS, W1, b1, W2, b2, W3):
    def body(m_r, rs_r, w1_r, b1_r, w2_r, b2_r, w3_r, o_r):
        m = m_r[...]
        r = rs_r[...]
        num = m[0, :, :64] + m[1, :, :64]
        rs = (r[0, :, 0] + r[1, :, 0])[:, None]
        h = num / (rs + 1e-16)
        h = jnp.maximum(jnp.dot(h, w1_r[...],
                                preferred_element_type=jnp.float32)
                        + b1_r[...], 0.0)
        h = jnp.maximum(jnp.dot(h, w2_r[...],
                                preferred_element_type=jnp.float32)
                        + b2_r[...], 0.0)
        lo = jnp.dot(h, w3_r[...], preferred_element_type=jnp.float32)
        mx = jnp.max(lo, axis=1, keepdims=True)
        ex = jnp.exp(lo - mx)
        o_r[...] = ex / jnp.sum(ex, axis=1, keepdims=True)
    specs = [pl.BlockSpec((2, RB, 128), lambda i: (0, i, 0)),
             pl.BlockSpec((2, RB, 1), lambda i: (0, i, 0)),
             _full((64, 64)), _full((1, 64)), _full((64, 64)),
             _full((1, 64)), _full((64, 16))]
    return _node_call(body, 1, [16], specs)(MO, RS, W1, b1, W2, b2, W3)


def _tc_edge_mean(ep):
    ep3 = ep.reshape(2, EPAD // 128, 128)

    def body(e_r, o_r):
        e = e_r[...]
        o_r[...] = (e[0] + e[1]) * 0.25
    return pl.pallas_call(
        body, grid=(1,),
        in_specs=[pl.BlockSpec((2, EPAD // 128, 128), lambda i: (0, 0, 0))],
        out_specs=pl.BlockSpec((EPAD // 128, 128), lambda i: (0, 0)),
        out_shape=jax.ShapeDtypeStruct((EPAD // 128, 128), jnp.float32),
    )(ep3)


def _head_tables(W, a):
    P = jnp.einsum('hef,hf->eh', W, a[:, :64])
    Q = jnp.einsum('hef,hf->eh', W, a[:, 64:128])
    return jnp.concatenate([P, Q], axis=1), a[:, 128]


def _single_tables(a):
    PQ = jnp.zeros((64, 8), jnp.float32)
    PQ = PQ.at[:, 0].set(a[:64]).at[:, 4].set(a[64:128])
    return PQ, a[128]


def _multi_sc_tables(SS, cvec):
    ssT = SS.T
    sstab = jnp.stack([
        jnp.stack([ssT[0], ssT[1], ssT[4], ssT[5]]),
        jnp.stack([ssT[2], ssT[3], ssT[6], ssT[7]]),
    ])
    cb = jnp.broadcast_to(cvec.reshape(2, 2, 1), (2, 2, 16))
    return sstab, cb


def _single_sc_tables(SS, cval):
    ssT = SS.T
    one = jnp.stack([ssT[0], ssT[4]])
    sstab = jnp.stack([one, one])
    cb = jnp.zeros((2, 2, 16), jnp.float32).at[:, 0, :].set(cval)
    return sstab, cb


def _pad_edges(edge):
    src = edge[0].astype(jnp.int32)
    dst = edge[1].astype(jnp.int32)
    pad = EPAD - E
    fill = (jnp.arange(pad, dtype=jnp.int32) * 37) % N
    return (jnp.concatenate([src, fill]), jnp.concatenate([dst, fill]))


def kernel(x, edgeA, edgeB, edge_feat, W_in, b_in, Wu, au, Wv, av,
           Wou, aou, Wov, aov, W1, b1, W2, b2, W3):
    ef = edge_feat[:, 0]
    efA = jnp.concatenate([ef, jnp.zeros((EPAD - E,), jnp.float32)])
    srcA, dstA = _pad_edges(edgeA)
    srcB, dstB = _pad_edges(edgeB)

    PQA, cvA = _head_tables(Wu, au)
    PQB, cvB = _head_tables(Wv, av)
    PQoA, coA = _single_tables(aou)
    PQoB, coB = _single_tables(aov)

    Z2 = jnp.zeros((128, 128), jnp.float32)
    Z1 = jnp.zeros((NPAD * 2 // NS,), jnp.float32)

    xp = jnp.pad(x, ((0, NPAD - N), (0, 0)))
    h0, SSA = _tc_stage0(xp, W_in, b_in.reshape(1, -1), PQA)
    ssA, cbA = _multi_sc_tables(SSA, cvA)
    MA, RSA, epA = _edge_multi(srcA, dstA, efA, h0, ssA, cbA, Z2, Z1)
    G2, SSoA = _tc_heads(MA, RSA.reshape(2, NPAD, 2), Wu, Wou, PQoA)
    ssoA, cboA = _single_sc_tables(SSoA, coA)
    MOA, RSoA = _edge_single(srcA, dstA, efA, G2, ssoA, cboA, Z2, Z1)
    h1, SSB = _tc_combine(MOA, RSoA.reshape(2, NPAD, 1), PQB)
    neA = _tc_edge_mean(epA).reshape(EPAD)

    ssB, cbB = _multi_sc_tables(SSB, cvB)
    MB, RSB, epB = _edge_multi(srcB, dstB, neA, h1, ssB, cbB, Z2, Z1)
    G4, SSoB = _tc_heads(MB, RSB.reshape(2, NPAD, 2), Wv, Wov, PQoB)
    ssoB, cboB = _single_sc_tables(SSoB, coB)
    MOB, RSoB = _edge_single(srcB, dstB, neA, G4, ssoB, cboB, Z2, Z1)
    out = _tc_final(MOB, RSoB.reshape(2, NPAD, 1),
                    W1, b1.reshape(1, -1), W2, b2.reshape(1, -1), W3)
    neB = _tc_edge_mean(epB).reshape(EPAD)

    return out[:N], neB[:E].reshape(E, 1)


# speedup vs baseline: 11.2354x; 1.3291x over previous
import functools

import jax
import jax.numpy as jnp
from jax import lax
from jax.experimental import pallas as pl
from jax.experimental.pallas import tpu as pltpu
from jax.experimental.pallas import tpu_sc as plsc

N = 10000
NPAD = 10240
E = 320000
EPAD = 327680
NC = 2
NS = 16
RPT = NPAD // NS
ALPHA = 0.2


W = 128


def _make_edge_pass(K):
    ECHUNK = EPAD // NS if K == 2 else EPAD // (NC * NS)
    NWIN = ECHUNK // W
    RSN = NPAD * K

    mesh = plsc.VectorSubcoreMesh(core_axis_name="c", subcore_axis_name="s")

    out_type = [jax.ShapeDtypeStruct((NC, NPAD, 128), jnp.float32),
                jax.ShapeDtypeStruct((NC, RSN), jnp.float32)]
    if K == 2:
        out_type.append(jax.ShapeDtypeStruct((NC, EPAD), jnp.float32))

    scratch = [
        pltpu.VMEM_SHARED((NPAD, 128), jnp.float32),
        pltpu.VMEM_SHARED((RSN,), jnp.float32),
        [pltpu.VMEM_SHARED((NPAD,), jnp.float32) for _ in range(2 * K)],
        pltpu.VMEM((3, W), jnp.int32),
        pltpu.VMEM((W, 128), jnp.float32),
        [pltpu.VMEM((W,), jnp.float32) for _ in range(2 * K)],
        pltpu.VMEM((W, 128), jnp.float32),
        pltpu.VMEM((W,), jnp.int32),
        [pltpu.VMEM((W,), jnp.float32) for _ in range(K)],
        [pltpu.VMEM((W,), jnp.int32) for _ in range(K)],
        pltpu.VMEM((W,), jnp.float32),
        pltpu.VMEM((2, 16), jnp.float32),
        pltpu.SemaphoreType.DMA,
    ]

    @functools.partial(pl.kernel, out_type=out_type, scratch_types=scratch,
                       mesh=mesh)
    def edge_pass(epk_hbm, g_hbm, ss_hbm, cb_hbm, z2_hbm, z1_hbm,
                  m_out, rs_out, *rest):
        if K == 2:
            ep_out = rest[0]
            rest = rest[1:]
        (m_sp, rs_sp, ss_sps, epk_v, g_v, ss_vs, pay_v, six_v,
         e_vs, ix_vs, ep_v, cb_v, sem) = rest

        c = lax.axis_index("c")
        s = lax.axis_index("s")
        r0 = s * RPT

        for t in range(2 * K):
            pltpu.sync_copy(ss_hbm.at[c, t, pl.ds(r0, RPT)],
                            ss_sps[t].at[pl.ds(r0, RPT)])
        for j in range(5):
            pltpu.sync_copy(z2_hbm.at[pl.ds(0, 128)],
                            m_sp.at[pl.ds(r0 + j * 128, 128)])
        pltpu.sync_copy(z1_hbm.at[pl.ds(0, RSN // NS)],
                        rs_sp.at[pl.ds(s * (RSN // NS), RSN // NS)])
        pltpu.sync_copy(z2_hbm.at[pl.ds(0, W)], pay_v)
        pltpu.sync_copy(cb_hbm.at[c], cb_v)
        plsc.subcore_barrier()

        cvals = [cb_v[h] for h in range(K)]
        iota16 = lax.iota(jnp.int32, 16)
        wb0 = (s * NWIN) if K == 2 else ((c * NS + s) * NWIN)

        def body(w, carry):
            eb = (wb0 + w) * W
            pltpu.sync_copy(epk_hbm.at[:, pl.ds(eb, W)], epk_v)
            gh = pltpu.async_copy(g_hbm.at[epk_v.at[1]], g_v, sem)
            for t in range(K):
                pltpu.sync_copy(ss_sps[t].at[epk_v.at[0]], ss_vs[t])
                pltpu.sync_copy(ss_sps[K + t].at[epk_v.at[1]], ss_vs[K + t])
            for g16 in range(W // 16):
                o = g16 * 16
                efv = lax.bitcast_convert_type(epk_v[2, pl.ds(o, 16)],
                                               jnp.float32)
                gid = (eb + o) + iota16
                valid = gid < E
                esum = jnp.zeros((16,), jnp.float32)
                src16 = epk_v[0, pl.ds(o, 16)]
                six_v[pl.ds(o, 16)] = src16
                for h in range(K):
                    s1 = ss_vs[h][pl.ds(o, 16)]
                    s2 = ss_vs[K + h][pl.ds(o, 16)]
                    lg = s1 + s2 + cvals[h] * efv
                    e = jnp.exp(-jnp.maximum(lg, ALPHA * lg))
                    e = jnp.where(valid, e, jnp.zeros((16,), jnp.float32))
                    esum = esum + e
                    e_vs[h][pl.ds(o, 16)] = e
                    if K == 2:
                        ix_vs[h][pl.ds(o, 16)] = src16 * 2 + h
                if K == 2:
                    ep_v[pl.ds(o, 16)] = esum
            gh.wait()
            for g16 in range(W // 16):
                o = g16 * 16
                evals = [e_vs[h][pl.ds(o, 16)] for h in range(K)]
                for j in range(16):
                    w2 = o + j
                    gq = [g_v[w2, pl.ds(q * 16, 16)] for q in range(4)]
                    for h in range(K):
                        ev = jnp.full((16,), evals[h][j])
                        for q in range(4):
                            pay_v[w2, pl.ds(h * 64 + q * 16, 16)] = (
                                ev * gq[q])
            pltpu.sync_copy(pay_v, m_sp.at[six_v], add=True)
            if K == 2:
                for h in range(K):
                    pltpu.sync_copy(e_vs[h], rs_sp.at[ix_vs[h]], add=True)
                pltpu.sync_copy(ep_v, ep_out.at[c, pl.ds(eb, W)])
            else:
                pltpu.sync_copy(e_vs[0], rs_sp.at[six_v], add=True)
            return carry

        lax.fori_loop(0, NWIN, body, 0)
        plsc.subcore_barrier()
        pltpu.sync_copy(m_sp.at[pl.ds(r0, RPT)], m_out.at[c, pl.ds(r0, RPT)])
        pltpu.sync_copy(rs_sp.at[pl.ds(s * (RSN // NS), RSN // NS)],
                        rs_out.at[c, pl.ds(s * (RSN // NS), RSN // NS)])

    return edge_pass


_edge_multi = _make_edge_pass(2)
_edge_single = _make_edge_pass(1)


RB = 2048


def _node_call(body, nout, out_widths, in_specs, out_dtype=jnp.float32):
    grid = (NPAD // RB,)
    out_specs = [pl.BlockSpec((RB, w), lambda i: (i, 0)) for w in out_widths]
    out_shape = [jax.ShapeDtypeStruct((NPAD, w), out_dtype)
                 for w in out_widths]
    if nout == 1:
        out_specs, out_shape = out_specs[0], out_shape[0]
    return pl.pallas_call(body, grid=grid, in_specs=in_specs,
                          out_specs=out_specs, out_shape=out_shape)


def _full(shape):
    return pl.BlockSpec(shape, lambda i: tuple(0 for _ in shape))


def _tc_stage0(x, W_in, b_in, PQ):
    def body(x_r, w_r, b_r, pq_r, h_r, ss_r):
        h = jnp.dot(x_r[...], w_r[...],
                    preferred_element_type=jnp.float32) + b_r[...]
        h_r[...] = jnp.concatenate([h, jnp.zeros((RB, 64), jnp.float32)], 1)
        ss_r[...] = jnp.dot(h, pq_r[...], preferred_element_type=jnp.float32)
    specs = [pl.BlockSpec((RB, 128), lambda i: (i, 0)), _full((128, 64)),
             _full((1, 64)), _full((64, 8))]
    return _node_call(body, 2, [128, 8], specs)(x, W_in, b_in, PQ)


def _tc_heads(M, RS, Wh, Wo, PQ):
    def body(m_r, rs_r, wh_r, wo_r, pq_r, g_r, ss_r):
        m = m_r[...]
        r = rs_r[...]
        wh = wh_r[...]
        wo = wo_r[...]
        acc = jnp.zeros((RB, 64), jnp.float32)
        for i in range(4):
            ci, li = divmod(i, 2)
            Mi = m[ci, :, li * 64:(li + 1) * 64]
            rs = r[ci, :, li][:, None]
            hp = jnp.dot(Mi / (rs + 1e-16), wh[i],
                         preferred_element_type=jnp.float32)
            el = jnp.where(hp > 0, hp, jnp.exp(hp) - 1.0)
            acc = acc + jnp.dot(el, wo[i * 64:(i + 1) * 64],
                                preferred_element_type=jnp.float32)
        g_r[...] = jnp.concatenate([acc, jnp.zeros((RB, 64), jnp.float32)], 1)
        ss_r[...] = jnp.dot(acc, pq_r[...], preferred_element_type=jnp.float32)
    specs = [pl.BlockSpec((2, RB, 128), lambda i: (0, i, 0)),
             pl.BlockSpec((2, RB, 2), lambda i: (0, i, 0)),
             _full((4, 64, 64)), _full((256, 64)), _full((64, 8))]
    return _node_call(body, 2, [128, 8], specs)(M, RS, Wh, Wo, PQ)


def _tc_combine(MO, RS, PQ):
    def body(m_r, rs_r, pq_r, h_r, ss_r):
        m = m_r[...]
        r = rs_r[...]
        num = m[0, :, :64] + m[1, :, :64]
        rs = (r[0, :, 0] + r[1, :, 0])[:, None]
        h = num / (rs + 1e-16)
        h_r[...] = jnp.concatenate([h, jnp.zeros((RB, 64), jnp.float32)], 1)
        ss_r[...] = jnp.dot(h, pq_r[...], preferred_element_type=jnp.float32)
    specs = [pl.BlockSpec((2, RB, 128), lambda i: (0, i, 0)),
             pl.BlockSpec((2, RB, 1), lambda i: (0, i, 0)), _full((64, 8))]
    return _node_call(body, 2, [128, 8], specs)(MO, RS, PQ)


def _tc_final(MO, RS, W1, b1, W2, b2, W3):
    def body(m_r, rs_r, w1_r, b1_r, w2_r, b2_r, w3_r, o_r):
        m = m_r[...]
        r = rs_r[...]
        num = m[0, :, :64] + m[1, :, :64]
        rs = (r[0, :, 0] + r[1, :, 0])[:, None]
        h = num / (rs + 1e-16)
        h = jnp.maximum(jnp.dot(h, w1_r[...],
                                preferred_element_type=jnp.float32)
                        + b1_r[...], 0.0)
        h = jnp.maximum(jnp.dot(h, w2_r[...],
                                preferred_element_type=jnp.float32)
                        + b2_r[...], 0.0)
        lo = jnp.dot(h, w3_r[...], preferred_element_type=jnp.float32)
        mx = jnp.max(lo, axis=1, keepdims=True)
        ex = jnp.exp(lo - mx)
        o_r[...] = ex / jnp.sum(ex, axis=1, keepdims=True)
    specs = [pl.BlockSpec((2, RB, 128), lambda i: (0, i, 0)),
             pl.BlockSpec((2, RB, 1), lambda i: (0, i, 0)),
             _full((64, 64)), _full((1, 64)), _full((64, 64)),
             _full((1, 64)), _full((64, 16))]
    return _node_call(body, 1, [16], specs)(MO, RS, W1, b1, W2, b2, W3)


def _tc_edge_mean(ep):
    ep3 = ep.reshape(2, EPAD // 128, 128)

    def body(e_r, o_r):
        e = e_r[...]
        o_r[...] = (e[0] + e[1]) * 0.25
    return pl.pallas_call(
        body, grid=(1,),
        in_specs=[pl.BlockSpec((2, EPAD // 128, 128), lambda i: (0, 0, 0))],
        out_specs=pl.BlockSpec((EPAD // 128, 128), lambda i: (0, 0)),
        out_shape=jax.ShapeDtypeStruct((EPAD // 128, 128), jnp.float32),
    )(ep3)


def _head_tables(W, a):
    P = jnp.einsum('hef,hf->eh', W, a[:, :64])
    Q = jnp.einsum('hef,hf->eh', W, a[:, 64:128])
    return jnp.concatenate([P, Q], axis=1), a[:, 128]


def _single_tables(a):
    PQ = jnp.zeros((64, 8), jnp.float32)
    PQ = PQ.at[:, 0].set(a[:64]).at[:, 4].set(a[64:128])
    return PQ, a[128]


def _multi_sc_tables(SS, cvec):
    ssT = SS.T
    sstab = jnp.stack([
        jnp.stack([ssT[0], ssT[1], ssT[4], ssT[5]]),
        jnp.stack([ssT[2], ssT[3], ssT[6], ssT[7]]),
    ])
    cb = jnp.broadcast_to(cvec.reshape(2, 2, 1), (2, 2, 16))
    return sstab, cb


def _single_sc_tables(SS, cval):
    ssT = SS.T
    one = jnp.stack([ssT[0], ssT[4]])
    sstab = jnp.stack([one, one])
    cb = jnp.zeros((2, 2, 16), jnp.float32).at[:, 0, :].set(cval)
    return sstab, cb


def _pack_edges(edge, ef):
    src = edge[0].astype(jnp.int32)
    dst = edge[1].astype(jnp.int32)
    pad = EPAD - E
    fill = (jnp.arange(pad, dtype=jnp.int32) * 37) % N
    return jnp.stack([
        jnp.concatenate([src, fill]),
        jnp.concatenate([dst, fill]),
        jax.lax.bitcast_convert_type(
            jnp.concatenate([ef, jnp.zeros((pad,), jnp.float32)]), jnp.int32),
    ])


def _repack_ef(epk, ne):
    return jnp.concatenate(
        [epk[:2], jax.lax.bitcast_convert_type(ne, jnp.int32)[None]], axis=0)


def kernel(x, edgeA, edgeB, edge_feat, W_in, b_in, Wu, au, Wv, av,
           Wou, aou, Wov, aov, W1, b1, W2, b2, W3):
    ef = edge_feat[:, 0]
    epkA = _pack_edges(edgeA, ef)
    epkB0 = _pack_edges(edgeB, ef)

    PQA, cvA = _head_tables(Wu, au)
    PQB, cvB = _head_tables(Wv, av)
    PQoA, coA = _single_tables(aou)
    PQoB, coB = _single_tables(aov)

    Z2 = jnp.zeros((128, 128), jnp.float32)
    Z1 = jnp.zeros((NPAD * 2 // NS,), jnp.float32)

    xp = jnp.pad(x, ((0, NPAD - N), (0, 0)))
    h0, SSA = _tc_stage0(xp, W_in, b_in.reshape(1, -1), PQA)
    ssA, cbA = _multi_sc_tables(SSA, cvA)
    MA, RSA, epA = _edge_multi(epkA, h0, ssA, cbA, Z2, Z1)
    G2, SSoA = _tc_heads(MA, RSA.reshape(2, NPAD, 2), Wu, Wou, PQoA)
    ssoA, cboA = _single_sc_tables(SSoA, coA)
    MOA, RSoA = _edge_single(epkA, G2, ssoA, cboA, Z2, Z1)
    h1, SSB = _tc_combine(MOA, RSoA.reshape(2, NPAD, 1), PQB)
    neA = _tc_edge_mean(epA).reshape(EPAD)

    epkB = _repack_ef(epkB0, neA)
    ssB, cbB = _multi_sc_tables(SSB, cvB)
    MB, RSB, epB = _edge_multi(epkB, h1, ssB, cbB, Z2, Z1)
    G4, SSoB = _tc_heads(MB, RSB.reshape(2, NPAD, 2), Wv, Wov, PQoB)
    ssoB, cboB = _single_sc_tables(SSoB, coB)
    MOB, RSoB = _edge_single(epkB, G4, ssoB, cboB, Z2, Z1)
    out = _tc_final(MOB, RSoB.reshape(2, NPAD, 1),
                    W1, b1.reshape(1, -1), W2, b2.reshape(1, -1), W3)
    neB = _tc_edge_mean(epB).reshape(EPAD)

    return out[:N], neB[:E].reshape(E, 1)
